# initial kernel scaffold (unmeasured)
import jax
import jax.numpy as jnp
from jax import lax
from jax.experimental import pallas as pl
from jax.experimental.pallas import tpu as pltpu

N_DEV = 4


def kernel(x, router_W, route_idx, expert_W, shared_W):
    n_tok, d = x.shape
    e_loc, _, h = expert_W.shape

    def body(x_ref, router_ref, idx_ref, expert_ref, shared_ref,
             out_ref, comm_ref, send_sems, recv_sems):
        my_i = lax.axis_index("i")
        left = lax.rem(my_i - 1 + N_DEV, N_DEV)
        right = lax.rem(my_i + 1, N_DEV)

        barrier_sem = pltpu.get_barrier_semaphore()
        for nbr in (left, right):
            pl.semaphore_signal(
                barrier_sem, inc=1,
                device_id=(nbr,), device_id_type=pl.DeviceIdType.MESH,
            )
        pl.semaphore_wait(barrier_sem, 2)

        xv = x_ref[:, :]

        scores = jnp.dot(xv, router_ref[:, :],
                         preferred_element_type=jnp.float32)
        s_max = jnp.max(scores, axis=-1, keepdims=True)
        p = jnp.exp(scores - s_max)
        probs = p / jnp.sum(p, axis=-1, keepdims=True)

        e_col = idx_ref[:, :]
        expert_iota = lax.broadcasted_iota(jnp.int32, scores.shape, 1)
        onehot = (expert_iota == e_col).astype(jnp.float32)
        p_sel = jnp.sum(probs * onehot, axis=-1, keepdims=True)

        own = jnp.zeros((n_tok, h), jnp.float32)
        for k in range(e_loc):
            g_e = my_i * e_loc + k
            gate = p_sel * (e_col == g_e).astype(jnp.float32)
            own = own + jnp.dot(xv * gate, expert_ref[k],
                                preferred_element_type=jnp.float32)

        comm_ref[0, :, :] = own
        shared_out = jnp.dot(xv, shared_ref[:, :],
                             preferred_element_type=jnp.float32)
        out_ref[:, :] = shared_out + own

        for hp in range(N_DEV - 1):
            s_slot = hp % 2
            r_slot = (hp + 1) % 2
            rdma = pltpu.make_async_remote_copy(
                src_ref=comm_ref.at[s_slot],
                dst_ref=comm_ref.at[r_slot],
                send_sem=send_sems.at[s_slot],
                recv_sem=recv_sems.at[r_slot],
                device_id=(right,),
                device_id_type=pl.DeviceIdType.MESH,
            )
            rdma.start()
            rdma.wait()
            out_ref[:, :] = out_ref[:, :] + comm_ref[r_slot, :, :]

    return pl.pallas_call(
        body,
        out_shape=jax.ShapeDtypeStruct((n_tok, h), jnp.float32),
        in_specs=[
            pl.BlockSpec(memory_space=pltpu.VMEM),
            pl.BlockSpec(memory_space=pltpu.VMEM),
            pl.BlockSpec(memory_space=pltpu.VMEM),
            pl.BlockSpec(memory_space=pltpu.VMEM),
            pl.BlockSpec(memory_space=pltpu.VMEM),
        ],
        out_specs=pl.BlockSpec(memory_space=pltpu.VMEM),
        scratch_shapes=[
            pltpu.VMEM((2, n_tok, h), jnp.float32),
            pltpu.SemaphoreType.DMA((2,)),
            pltpu.SemaphoreType.DMA((2,)),
        ],
        compiler_params=pltpu.CompilerParams(collective_id=0),
    )(x, router_W, route_idx, expert_W, shared_W)


# baseline (device time: 315842 ns/iter reference)
import jax
import jax.numpy as jnp
from jax import lax
from jax.experimental import pallas as pl
from jax.experimental.pallas import tpu as pltpu

N_DEV = 4


def kernel(x, router_W, route_idx, expert_W, shared_W):
    n_tok, d = x.shape
    e_loc, _, h = expert_W.shape

    def body(x_ref, router_ref, idx_ref, expert_ref, shared_ref,
             out_ref, comm_ref, send_sems, recv_sems):
        my_i = lax.axis_index("i")
        left = lax.rem(my_i - 1 + N_DEV, N_DEV)
        right = lax.rem(my_i + 1, N_DEV)

        barrier_sem = pltpu.get_barrier_semaphore()
        for nbr in (left, right):
            pl.semaphore_signal(
                barrier_sem, inc=1,
                device_id=(nbr,), device_id_type=pl.DeviceIdType.MESH,
            )
        pl.semaphore_wait(barrier_sem, 2)

        n_chunks = 4
        rows = n_tok // n_chunks
        for c in range(n_chunks):
            sl = pl.ds(c * rows, rows)
            xc = x_ref[sl, :]

            scores = jnp.dot(xc, router_ref[:, :],
                             preferred_element_type=jnp.float32)
            s_max = jnp.max(scores, axis=-1, keepdims=True)
            p = jnp.exp(scores - s_max)
            probs = p / jnp.sum(p, axis=-1, keepdims=True)

            e_col = idx_ref[sl, :]
            expert_iota = lax.broadcasted_iota(jnp.int32, scores.shape, 1)
            onehot = (expert_iota == e_col).astype(jnp.float32)
            p_sel = jnp.sum(probs * onehot, axis=-1, keepdims=True)

            own = jnp.zeros((rows, h), jnp.float32)
            for k in range(e_loc):
                g_e = my_i * e_loc + k
                gate = p_sel * (e_col == g_e).astype(jnp.float32)
                own = own + jnp.dot(xc * gate, expert_ref[k],
                                    preferred_element_type=jnp.float32)

            comm_ref[0, sl, :] = own
            shared_out = jnp.dot(xc, shared_ref[:, :],
                                 preferred_element_type=jnp.float32)
            out_ref[sl, :] = shared_out
            out_ref[sl, :] = out_ref[sl, :] + comm_ref[0, sl, :]

        for hp in range(N_DEV - 1):
            s_slot = hp % 2
            r_slot = (hp + 1) % 2
            rdma = pltpu.make_async_remote_copy(
                src_ref=comm_ref.at[s_slot],
                dst_ref=comm_ref.at[r_slot],
                send_sem=send_sems.at[s_slot],
                recv_sem=recv_sems.at[r_slot],
                device_id=(right,),
                device_id_type=pl.DeviceIdType.MESH,
            )
            rdma.start()
            rdma.wait()
            out_ref[:, :] = out_ref[:, :] + comm_ref[r_slot, :, :]

    return pl.pallas_call(
        body,
        out_shape=jax.ShapeDtypeStruct((n_tok, h), jnp.float32),
        in_specs=[
            pl.BlockSpec(memory_space=pltpu.VMEM),
            pl.BlockSpec(memory_space=pltpu.VMEM),
            pl.BlockSpec(memory_space=pltpu.VMEM),
            pl.BlockSpec(memory_space=pltpu.VMEM),
            pl.BlockSpec(memory_space=pltpu.VMEM),
        ],
        out_specs=pl.BlockSpec(memory_space=pltpu.VMEM),
        scratch_shapes=[
            pltpu.VMEM((2, n_tok, h), jnp.float32),
            pltpu.SemaphoreType.DMA((2,)),
            pltpu.SemaphoreType.DMA((2,)),
        ],
        compiler_params=pltpu.CompilerParams(
            collective_id=0,
            vmem_limit_bytes=100 * 1024 * 1024,
        ),
    )(x, router_W, route_idx, expert_W, shared_W)


# device time: 171783 ns/iter; 1.8386x vs baseline; 1.8386x over previous
import jax
import jax.numpy as jnp
from jax import lax
from jax.experimental import pallas as pl
from jax.experimental.pallas import tpu as pltpu

N_DEV = 4


def kernel(x, router_W, route_idx, expert_W, shared_W):
    n_tok, d = x.shape
    e_loc, _, h = expert_W.shape
    ch = n_tok // N_DEV

    def body(x_ref, router_ref, idx_ref, expert_ref, shared_ref,
             out_ref, buf, agbuf, pbuf, send_sems, recv_sems):
        my_i = lax.axis_index("i")
        left = lax.rem(my_i + N_DEV - 1, N_DEV)
        right = lax.rem(my_i + 1, N_DEV)

        barrier_sem = pltpu.get_barrier_semaphore()
        for nbr in (left, right):
            pl.semaphore_signal(
                barrier_sem, inc=1,
                device_id=(nbr,), device_id_type=pl.DeviceIdType.MESH,
            )
        pl.semaphore_wait(barrier_sem, 2)

        def partial_store(dst):
            def f(c):
                sl = pl.ds(c * ch, ch)
                xc = x_ref[sl, :]
                scores = jnp.dot(xc, router_ref[:, :],
                                 preferred_element_type=jnp.float32)
                s_max = jnp.max(scores, axis=-1, keepdims=True)
                p = jnp.exp(scores - s_max)
                probs = p / jnp.sum(p, axis=-1, keepdims=True)
                e_col = idx_ref[sl, :]
                iota = lax.broadcasted_iota(jnp.int32, scores.shape, 1)
                onehot = (iota == e_col).astype(jnp.float32)
                p_sel = jnp.sum(probs * onehot, axis=-1, keepdims=True)
                own = jnp.zeros((ch, h), jnp.float32)
                for k in range(e_loc):
                    g_e = my_i * e_loc + k
                    gate = p_sel * (e_col == g_e).astype(jnp.float32)
                    own = own + jnp.dot(xc * gate, expert_ref[k],
                                        preferred_element_type=jnp.float32)
                dst[:, :] = own
            return f

        rdmas = []

        partial_store(buf.at[0])(my_i)
        for s in range(N_DEV - 1):
            rdma = pltpu.make_async_remote_copy(
                src_ref=buf.at[s],
                dst_ref=buf.at[s + 1],
                send_sem=send_sems.at[s],
                recv_sem=recv_sems.at[s],
                device_id=(right,),
                device_id_type=pl.DeviceIdType.MESH,
            )
            rdma.start()
            rdmas.append(rdma)
            c_in = lax.rem(my_i - s - 1 + 2 * N_DEV, N_DEV)
            partial_store(pbuf)(c_in)
            rdma.wait_recv()
            buf[s + 1, :, :] = buf[s + 1, :, :] + pbuf[:, :]

        c_own = lax.rem(my_i + 1, N_DEV)
        for t in range(N_DEV - 1):
            src = buf.at[N_DEV - 1] if t == 0 else agbuf.at[t - 1]
            rdma = pltpu.make_async_remote_copy(
                src_ref=src,
                dst_ref=agbuf.at[t],
                send_sem=send_sems.at[N_DEV - 1 + t],
                recv_sem=recv_sems.at[N_DEV - 1 + t],
                device_id=(right,),
                device_id_type=pl.DeviceIdType.MESH,
            )
            rdma.start()
            rdmas.append(rdma)
            if t == 0:
                sl_own = pl.ds(c_own * ch, ch)
                out_ref[sl_own, :] = jnp.dot(x_ref[sl_own, :], shared_ref[:, :],
                                             preferred_element_type=jnp.float32)
                out_ref[sl_own, :] = out_ref[sl_own, :] + buf[N_DEV - 1, :, :]
            a_t = lax.rem(my_i - t + 2 * N_DEV, N_DEV)
            sl_a = pl.ds(a_t * ch, ch)
            out_ref[sl_a, :] = jnp.dot(x_ref[sl_a, :], shared_ref[:, :],
                                       preferred_element_type=jnp.float32)
            rdma.wait_recv()
            out_ref[sl_a, :] = out_ref[sl_a, :] + agbuf[t, :, :]

        for rdma in rdmas:
            rdma.wait_send()

    return pl.pallas_call(
        body,
        out_shape=jax.ShapeDtypeStruct((n_tok, h), jnp.float32),
        in_specs=[pl.BlockSpec(memory_space=pltpu.VMEM)] * 5,
        out_specs=pl.BlockSpec(memory_space=pltpu.VMEM),
        scratch_shapes=[
            pltpu.VMEM((N_DEV, ch, h), jnp.float32),
            pltpu.VMEM((N_DEV - 1, ch, h), jnp.float32),
            pltpu.VMEM((ch, h), jnp.float32),
            pltpu.SemaphoreType.DMA((2 * (N_DEV - 1),)),
            pltpu.SemaphoreType.DMA((2 * (N_DEV - 1),)),
        ],
        compiler_params=pltpu.CompilerParams(
            collective_id=0,
            vmem_limit_bytes=100 * 1024 * 1024,
        ),
    )(x, router_W, route_idx, expert_W, shared_W)
